# baseline (speedup 1.0000x reference)
import functools
import math

import jax
import jax.numpy as jnp
import numpy as np
from jax.experimental import pallas as pl
from jax.experimental.pallas import tpu as pltpu

_MIN_SCALE = 1e-4
_EPS = 1e-8
_ALPHA = 0.5
_TINY = np.float32(np.finfo(np.float32).tiny)
_LOG2PI = np.float32(np.log(2.0 * np.pi))
_NEG_INF = np.float32(-np.inf)

_I_BLK = 2048
_ROWS = 8
_JW = 512
_BG = 8


def _tf_round(x0, x1, r):
    x0 = x0 + x1
    x1 = (x1 << np.uint32(r)) | (x1 >> np.uint32(32 - r))
    x1 = x1 ^ x0
    return x0, x1


def _bits_to_gumbel(bits):
    fb = (bits >> np.uint32(9)) | np.uint32(0x3F800000)
    f = jax.lax.bitcast_convert_type(fb, jnp.float32) - np.float32(1.0)
    u = jnp.maximum(_TINY, f + _TINY)
    return -jnp.log(-jnp.log(u))


def _stage_a(px0_ref, px1_ref, v0_ref, v1_ref, n0_ref, n1_ref, lw_ref,
             z_ref, anc_ref, sc_ref,
             oxp0_ref, oxp1_ref, ow_ref, oq_ref, ologit_ref, oest0_ref,
             oest1_ref, *, num_m):
    xp0 = px0_ref[...] + v0_ref[...] + n0_ref[...]
    xp1 = px1_ref[...] + v1_ref[...] + n1_ref[...]
    oxp0_ref[...] = xp0
    oxp1_ref[...] = xp1

    var = sc_ref[0]
    log_var = sc_ref[1]
    acc = jnp.zeros_like(xp0)
    for m in range(num_m):
        dx = xp0 - anc_ref[m, 0]
        dy = xp1 - anc_ref[m, 1]
        ypred = jnp.sqrt(dx * dx + dy * dy)
        zcol = jnp.concatenate(
            [jnp.full((1, 1), z_ref[bb, m], jnp.float32)
             for bb in range(_BG)], axis=0)
        inn = zcol - ypred
        acc = acc + (inn * inn / var + log_var + _LOG2PI)
    log_like = np.float32(-0.5) * acc

    lwu = lw_ref[...] + log_like
    amax = jnp.max(lwu, axis=-1, keepdims=True)
    lse = jnp.log(jnp.sum(jnp.exp(lwu - amax), axis=-1, keepdims=True)) + amax
    log_w = lwu - lse
    w = jnp.exp(log_w)
    ow_ref[...] = w

    num_p = xp0.shape[-1]
    oest0_ref[...] = jnp.broadcast_to(
        jnp.sum(w * xp0, axis=-1, keepdims=True), oest0_ref.shape)
    oest1_ref[...] = jnp.broadcast_to(
        jnp.sum(w * xp1, axis=-1, keepdims=True), oest1_ref.shape)

    q = np.float32(_ALPHA) * w + np.float32((1.0 - _ALPHA) * (1.0 / num_p))
    q_sum = jnp.sum(q, axis=-1, keepdims=True)
    safe_q = q / jnp.maximum(q_sum, np.float32(_EPS))
    safe_q = jnp.maximum(safe_q, np.float32(_EPS))
    safe_q = safe_q / jnp.maximum(
        jnp.sum(safe_q, axis=-1, keepdims=True), np.float32(_EPS))
    oq_ref[...] = safe_q
    ologit_ref[...] = jnp.log(safe_q)


def _stage_b(key_ref, logit_ref, xp0_ref, xp1_ref, w_ref, q_ref,
             oxr0_ref, oxr1_ref, owsel_ref, oqsel_ref, *, num_p, log2_n):
    b = pl.program_id(0)
    ib = pl.program_id(1)
    k0 = key_ref[0]
    k1 = key_ref[1]
    ks2 = k0 ^ k1 ^ np.uint32(0x1BD11BDA)

    n_tiles = _I_BLK // _ROWS
    n_jiters = num_p // _JW
    row0 = (b.astype(jnp.uint32) * np.uint32(num_p)
            + ib.astype(jnp.uint32) * np.uint32(_I_BLK))
    sub_iota = jax.lax.broadcasted_iota(jnp.uint32, (_ROWS, 1), 0)
    lane_iota_u = jax.lax.broadcasted_iota(jnp.uint32, (1, _JW), 1)
    lane_iota_i = jax.lax.broadcasted_iota(jnp.int32, (1, _JW), 1)
    lo_mask = np.uint32((1 << (32 - log2_n)) - 1)
    rot = ((13, 15, 26, 6), (17, 29, 16, 24))
    inj = ((k1, ks2), (ks2, k0), (k0, k1), (k1, ks2), (ks2, k0))

    def tile_body(t, _):
        r = row0 + t.astype(jnp.uint32) * np.uint32(_ROWS) + sub_iota
        hi = r >> np.uint32(32 - log2_n)
        lo_base = (r & lo_mask) << np.uint32(log2_n)
        x0_init = hi + k0

        def j_body(c, carry):
            best, bidx, bx0, bx1, bw, bq = carry
            jj_u = c.astype(jnp.uint32) * np.uint32(_JW) + lane_iota_u
            lo = lo_base + jj_u
            x0 = x0_init
            x1 = lo + k1
            for i in range(5):
                for rr in rot[i % 2]:
                    x0, x1 = _tf_round(x0, x1, rr)
                a, bb2 = inj[i]
                x0 = x0 + a
                x1 = x1 + bb2 + np.uint32(i + 1)
            g = _bits_to_gumbel(x0 ^ x1)
            score = g + logit_ref[0, c, :].reshape(1, _JW)
            upd = score > best
            jj_i = c * _JW + lane_iota_i
            best = jnp.where(upd, score, best)
            bidx = jnp.where(upd, jnp.broadcast_to(jj_i, upd.shape), bidx)
            bx0 = jnp.where(upd, xp0_ref[0, c, :].reshape(1, _JW), bx0)
            bx1 = jnp.where(upd, xp1_ref[0, c, :].reshape(1, _JW), bx1)
            bw = jnp.where(upd, w_ref[0, c, :].reshape(1, _JW), bw)
            bq = jnp.where(upd, q_ref[0, c, :].reshape(1, _JW), bq)
            return best, bidx, bx0, bx1, bw, bq

        init = (jnp.full((_ROWS, _JW), _NEG_INF, jnp.float32),
                jnp.zeros((_ROWS, _JW), jnp.int32),
                jnp.zeros((_ROWS, _JW), jnp.float32),
                jnp.zeros((_ROWS, _JW), jnp.float32),
                jnp.zeros((_ROWS, _JW), jnp.float32),
                jnp.zeros((_ROWS, _JW), jnp.float32))
        best, bidx, bx0, bx1, bw, bq = jax.lax.fori_loop(
            0, n_jiters, j_body, init)

        rowmax = jnp.max(best, axis=1, keepdims=True)
        widx = jnp.min(jnp.where(best == rowmax, bidx, np.int32(2**30)),
                       axis=1, keepdims=True)
        msk = bidx == widx
        vx0 = jnp.max(jnp.where(msk, bx0, _NEG_INF), axis=1)
        vx1 = jnp.max(jnp.where(msk, bx1, _NEG_INF), axis=1)
        vw = jnp.max(jnp.where(msk, bw, _NEG_INF), axis=1)
        vq = jnp.max(jnp.where(msk, bq, _NEG_INF), axis=1)
        oxr0_ref[0, t, :] = vx0
        oxr1_ref[0, t, :] = vx1
        owsel_ref[0, t, :] = vw
        oqsel_ref[0, t, :] = vq
        return 0

    jax.lax.fori_loop(0, n_tiles, tile_body, 0)


def _stage_c(wsel_ref, qsel_ref, xr0_ref, xr1_ref, px0_ref, px1_ref,
             olw_ref, ov0_ref, ov1_ref):
    w_corr = wsel_ref[...] / jnp.maximum(qsel_ref[...], np.float32(_EPS))
    lw = jnp.log(jnp.maximum(w_corr, np.float32(_EPS)))
    amax = jnp.max(lw, axis=-1, keepdims=True)
    lse = jnp.log(jnp.sum(jnp.exp(lw - amax), axis=-1, keepdims=True)) + amax
    olw_ref[...] = lw - lse
    ov0_ref[...] = xr0_ref[...] - px0_ref[...]
    ov1_ref[...] = xr1_ref[...] - px1_ref[...]


def kernel(x_prev, v_prev, log_w_prev, z_t, anchors, log_process_scale,
           log_obs_scale):
    bsz, num_p, _ = x_prev.shape
    num_m = z_t.shape[1]
    log2_n = int(math.log2(num_p))
    assert (1 << log2_n) == num_p

    key = jax.random.key(42)
    k_noise, k_res = jax.random.split(key)
    process_scale = jax.nn.softplus(log_process_scale) + _MIN_SCALE
    obs_scale = jax.nn.softplus(log_obs_scale) + _MIN_SCALE
    noise = jax.random.normal(k_noise, x_prev.shape, dtype=x_prev.dtype) \
        * process_scale.reshape(1, 1, -1)
    var = jnp.maximum(obs_scale * obs_scale, _MIN_SCALE)
    sc = jnp.concatenate([var, jnp.log(var)]).astype(jnp.float32)
    key_data = jax.random.key_data(k_res).astype(jnp.uint32)

    px0 = x_prev[:, :, 0]
    px1 = x_prev[:, :, 1]
    n_bg = bsz // _BG
    row = lambda: pl.BlockSpec((_BG, num_p), lambda g: (g, 0))
    est = lambda: pl.BlockSpec((_BG, 128), lambda g: (g, 0))

    xp0, xp1, w, safe_q, logits, est0, est1 = pl.pallas_call(
        functools.partial(_stage_a, num_m=num_m),
        grid=(n_bg,),
        in_specs=[row(), row(), row(), row(), row(), row(), row(),
                  pl.BlockSpec((_BG, num_m), lambda g: (g, 0),
                               memory_space=pltpu.SMEM),
                  pl.BlockSpec((num_m, 2), lambda g: (0, 0),
                               memory_space=pltpu.SMEM),
                  pl.BlockSpec((2,), lambda g: (0,),
                               memory_space=pltpu.SMEM)],
        out_specs=[row(), row(), row(), row(), row(), est(), est()],
        out_shape=[jax.ShapeDtypeStruct((bsz, num_p), jnp.float32)] * 5
        + [jax.ShapeDtypeStruct((bsz, 128), jnp.float32)] * 2,
        compiler_params=pltpu.CompilerParams(
            dimension_semantics=("parallel",)),
    )(px0, px1, v_prev[:, :, 0], v_prev[:, :, 1],
      noise[:, :, 0], noise[:, :, 1], log_w_prev, z_t, anchors, sc)

    n_iblk = num_p // _I_BLK
    n_jiters = num_p // _JW
    n_tiles = _I_BLK // _ROWS
    resh = lambda a: a.reshape(bsz, n_jiters, _JW)
    inb = lambda: pl.BlockSpec((1, n_jiters, _JW), lambda b, ib: (b, 0, 0))
    outb = lambda: pl.BlockSpec((1, n_tiles, _ROWS),
                                lambda b, ib: (b * n_iblk + ib, 0, 0))
    xr0, xr1, wsel, qsel = pl.pallas_call(
        functools.partial(_stage_b, num_p=num_p, log2_n=log2_n),
        grid=(bsz, n_iblk),
        in_specs=[pl.BlockSpec(memory_space=pltpu.SMEM),
                  inb(), inb(), inb(), inb(), inb()],
        out_specs=[outb(), outb(), outb(), outb()],
        out_shape=[jax.ShapeDtypeStruct((bsz * n_iblk, n_tiles, _ROWS),
                                        jnp.float32)] * 4,
        compiler_params=pltpu.CompilerParams(
            dimension_semantics=("parallel", "parallel")),
    )(key_data, resh(logits), resh(xp0), resh(xp1), resh(w), resh(safe_q))

    unb = lambda a: a.reshape(bsz, num_p)
    xr0, xr1, wsel, qsel = unb(xr0), unb(xr1), unb(wsel), unb(qsel)

    log_w_next, vn0, vn1 = pl.pallas_call(
        _stage_c,
        grid=(n_bg,),
        in_specs=[row(), row(), row(), row(), row(), row()],
        out_specs=[row(), row(), row()],
        out_shape=[jax.ShapeDtypeStruct((bsz, num_p), jnp.float32)] * 3,
        compiler_params=pltpu.CompilerParams(
            dimension_semantics=("parallel",)),
    )(wsel, qsel, xr0, xr1, px0, px1)

    x_next = jnp.stack([xr0, xr1], axis=-1)
    v_next = jnp.stack([vn0, vn1], axis=-1)
    x_est = jnp.stack([est0[:, 0], est1[:, 0]], axis=-1)
    return (x_next, v_next, log_w_next, x_est)


# speedup vs baseline: 1.4233x; 1.4233x over previous
import functools
import math

import jax
import jax.numpy as jnp
import numpy as np
from jax.experimental import pallas as pl
from jax.experimental.pallas import tpu as pltpu

_MIN_SCALE = 1e-4
_EPS = 1e-8
_ALPHA = 0.5
_TINY = np.float32(np.finfo(np.float32).tiny)
_LOG2PI = np.float32(np.log(2.0 * np.pi))
_NEG_INF = np.float32(-np.inf)

_I_BLK = 2048
_ROWS = 8
_JW = 1024
_BG = 8


def _tf_round(x0, x1, r):
    x0 = x0 + x1
    x1 = (x1 << np.uint32(r)) | (x1 >> np.uint32(32 - r))
    x1 = x1 ^ x0
    return x0, x1


def _bits_to_gumbel(bits):
    fb = (bits >> np.uint32(9)) | np.uint32(0x3F800000)
    f = jax.lax.bitcast_convert_type(fb, jnp.float32) - np.float32(1.0)
    u = jnp.maximum(_TINY, f + _TINY)
    return -jnp.log(-jnp.log(u))


def _stage_a(px0_ref, px1_ref, v0_ref, v1_ref, n0_ref, n1_ref, lw_ref,
             z_ref, anc_ref, sc_ref,
             oxp0_ref, oxp1_ref, ow_ref, oq_ref, ologit_ref, oest0_ref,
             oest1_ref, *, num_m):
    xp0 = px0_ref[...] + v0_ref[...] + n0_ref[...]
    xp1 = px1_ref[...] + v1_ref[...] + n1_ref[...]
    oxp0_ref[...] = xp0
    oxp1_ref[...] = xp1

    var = sc_ref[0]
    log_var = sc_ref[1]
    acc = jnp.zeros_like(xp0)
    for m in range(num_m):
        dx = xp0 - anc_ref[m, 0]
        dy = xp1 - anc_ref[m, 1]
        ypred = jnp.sqrt(dx * dx + dy * dy)
        zcol = jnp.concatenate(
            [jnp.full((1, 1), z_ref[bb, m], jnp.float32)
             for bb in range(_BG)], axis=0)
        inn = zcol - ypred
        acc = acc + (inn * inn / var + log_var + _LOG2PI)
    log_like = np.float32(-0.5) * acc

    lwu = lw_ref[...] + log_like
    amax = jnp.max(lwu, axis=-1, keepdims=True)
    lse = jnp.log(jnp.sum(jnp.exp(lwu - amax), axis=-1, keepdims=True)) + amax
    log_w = lwu - lse
    w = jnp.exp(log_w)
    ow_ref[...] = w

    num_p = xp0.shape[-1]
    oest0_ref[...] = jnp.broadcast_to(
        jnp.sum(w * xp0, axis=-1, keepdims=True), oest0_ref.shape)
    oest1_ref[...] = jnp.broadcast_to(
        jnp.sum(w * xp1, axis=-1, keepdims=True), oest1_ref.shape)

    q = np.float32(_ALPHA) * w + np.float32((1.0 - _ALPHA) * (1.0 / num_p))
    q_sum = jnp.sum(q, axis=-1, keepdims=True)
    safe_q = q / jnp.maximum(q_sum, np.float32(_EPS))
    safe_q = jnp.maximum(safe_q, np.float32(_EPS))
    safe_q = safe_q / jnp.maximum(
        jnp.sum(safe_q, axis=-1, keepdims=True), np.float32(_EPS))
    oq_ref[...] = safe_q
    ologit_ref[...] = jnp.log(safe_q)


def _stage_b(key_ref, logit_ref, xp0_ref, xp1_ref, w_ref, q_ref,
             oxr0_ref, oxr1_ref, owsel_ref, oqsel_ref, *, num_p, log2_n):
    b = pl.program_id(0)
    ib = pl.program_id(1)
    k0 = key_ref[0]
    k1 = key_ref[1]
    ks2 = k0 ^ k1 ^ np.uint32(0x1BD11BDA)

    n_tiles = _I_BLK // _ROWS
    n_jiters = num_p // _JW
    row0 = (b.astype(jnp.uint32) * np.uint32(num_p)
            + ib.astype(jnp.uint32) * np.uint32(_I_BLK))
    sub_iota = jax.lax.broadcasted_iota(jnp.uint32, (_ROWS, 1), 0)
    lane_iota_u = jax.lax.broadcasted_iota(jnp.uint32, (1, _JW), 1)
    lane_iota_i = jax.lax.broadcasted_iota(jnp.int32, (1, _JW), 1)
    lo_mask = np.uint32((1 << (32 - log2_n)) - 1)
    rot = ((13, 15, 26, 6), (17, 29, 16, 24))
    inj = ((k1, ks2), (ks2, k0), (k0, k1), (k1, ks2), (ks2, k0))

    def tile_body(t, _):
        r = row0 + t.astype(jnp.uint32) * np.uint32(_ROWS) + sub_iota
        hi = r >> np.uint32(32 - log2_n)
        lo_base = (r & lo_mask) << np.uint32(log2_n)
        x0_init = hi + k0
        x1_base = lo_base + k1

        def j_body(c, carry):
            best, bidx = carry
            jj_u = c.astype(jnp.uint32) * np.uint32(_JW) + lane_iota_u
            x0 = x0_init
            x1 = x1_base + jj_u
            for i in range(5):
                for rr in rot[i % 2]:
                    x0, x1 = _tf_round(x0, x1, rr)
                a, bb2 = inj[i]
                x0 = x0 + a
                x1 = x1 + bb2 + np.uint32(i + 1)
            bits = x0 ^ x1
            fb = (bits >> np.uint32(9)) | np.uint32(0x3F800000)
            f = jax.lax.bitcast_convert_type(fb, jnp.float32) - np.float32(1.0)
            u = jnp.maximum(_TINY, f + _TINY)
            nlu = np.float32(0.0) - jnp.log(u)
            score = logit_ref[0, c, :].reshape(1, _JW) - jnp.log(nlu)
            upd = score > best
            jj_i = c * _JW + lane_iota_i
            best = jnp.maximum(best, score)
            bidx = jnp.where(upd, jnp.broadcast_to(jj_i, upd.shape), bidx)
            return best, bidx

        init = (jnp.full((_ROWS, _JW), _NEG_INF, jnp.float32),
                jnp.zeros((_ROWS, _JW), jnp.int32))
        best, bidx = jax.lax.fori_loop(0, n_jiters, j_body, init)

        rowmax = jnp.max(best, axis=1, keepdims=True)
        widx = jnp.min(jnp.where(best == rowmax, bidx, np.int32(2**30)),
                       axis=1, keepdims=True)

        def v_body(c, acc):
            a0, a1, aw, aq = acc
            jj_i = c * _JW + lane_iota_i
            m = jj_i == widx
            a0 = jnp.where(m, xp0_ref[0, c, :].reshape(1, _JW), a0)
            a1 = jnp.where(m, xp1_ref[0, c, :].reshape(1, _JW), a1)
            aw = jnp.where(m, w_ref[0, c, :].reshape(1, _JW), aw)
            aq = jnp.where(m, q_ref[0, c, :].reshape(1, _JW), aq)
            return a0, a1, aw, aq

        z = jnp.zeros((_ROWS, _JW), jnp.float32)
        a0, a1, aw, aq = jax.lax.fori_loop(
            0, n_jiters, v_body, (z, z, z, z))
        oxr0_ref[0, t, :] = jnp.sum(a0, axis=1)
        oxr1_ref[0, t, :] = jnp.sum(a1, axis=1)
        owsel_ref[0, t, :] = jnp.sum(aw, axis=1)
        oqsel_ref[0, t, :] = jnp.sum(aq, axis=1)
        return 0

    jax.lax.fori_loop(0, n_tiles, tile_body, 0)


def _stage_c(wsel_ref, qsel_ref, xr0_ref, xr1_ref, px0_ref, px1_ref,
             olw_ref, ov0_ref, ov1_ref):
    w_corr = wsel_ref[...] / jnp.maximum(qsel_ref[...], np.float32(_EPS))
    lw = jnp.log(jnp.maximum(w_corr, np.float32(_EPS)))
    amax = jnp.max(lw, axis=-1, keepdims=True)
    lse = jnp.log(jnp.sum(jnp.exp(lw - amax), axis=-1, keepdims=True)) + amax
    olw_ref[...] = lw - lse
    ov0_ref[...] = xr0_ref[...] - px0_ref[...]
    ov1_ref[...] = xr1_ref[...] - px1_ref[...]


def kernel(x_prev, v_prev, log_w_prev, z_t, anchors, log_process_scale,
           log_obs_scale):
    bsz, num_p, _ = x_prev.shape
    num_m = z_t.shape[1]
    log2_n = int(math.log2(num_p))
    assert (1 << log2_n) == num_p

    key = jax.random.key(42)
    k_noise, k_res = jax.random.split(key)
    process_scale = jax.nn.softplus(log_process_scale) + _MIN_SCALE
    obs_scale = jax.nn.softplus(log_obs_scale) + _MIN_SCALE
    noise = jax.random.normal(k_noise, x_prev.shape, dtype=x_prev.dtype) \
        * process_scale.reshape(1, 1, -1)
    var = jnp.maximum(obs_scale * obs_scale, _MIN_SCALE)
    sc = jnp.concatenate([var, jnp.log(var)]).astype(jnp.float32)
    key_data = jax.random.key_data(k_res).astype(jnp.uint32)

    px0 = x_prev[:, :, 0]
    px1 = x_prev[:, :, 1]
    n_bg = bsz // _BG
    row = lambda: pl.BlockSpec((_BG, num_p), lambda g: (g, 0))
    est = lambda: pl.BlockSpec((_BG, 128), lambda g: (g, 0))

    xp0, xp1, w, safe_q, logits, est0, est1 = pl.pallas_call(
        functools.partial(_stage_a, num_m=num_m),
        grid=(n_bg,),
        in_specs=[row(), row(), row(), row(), row(), row(), row(),
                  pl.BlockSpec((_BG, num_m), lambda g: (g, 0),
                               memory_space=pltpu.SMEM),
                  pl.BlockSpec((num_m, 2), lambda g: (0, 0),
                               memory_space=pltpu.SMEM),
                  pl.BlockSpec((2,), lambda g: (0,),
                               memory_space=pltpu.SMEM)],
        out_specs=[row(), row(), row(), row(), row(), est(), est()],
        out_shape=[jax.ShapeDtypeStruct((bsz, num_p), jnp.float32)] * 5
        + [jax.ShapeDtypeStruct((bsz, 128), jnp.float32)] * 2,
        compiler_params=pltpu.CompilerParams(
            dimension_semantics=("parallel",)),
    )(px0, px1, v_prev[:, :, 0], v_prev[:, :, 1],
      noise[:, :, 0], noise[:, :, 1], log_w_prev, z_t, anchors, sc)

    n_iblk = num_p // _I_BLK
    n_jiters = num_p // _JW
    n_tiles = _I_BLK // _ROWS
    resh = lambda a: a.reshape(bsz, n_jiters, _JW)
    inb = lambda: pl.BlockSpec((1, n_jiters, _JW), lambda b, ib: (b, 0, 0))
    outb = lambda: pl.BlockSpec((1, n_tiles, _ROWS),
                                lambda b, ib: (b * n_iblk + ib, 0, 0))
    xr0, xr1, wsel, qsel = pl.pallas_call(
        functools.partial(_stage_b, num_p=num_p, log2_n=log2_n),
        grid=(bsz, n_iblk),
        in_specs=[pl.BlockSpec(memory_space=pltpu.SMEM),
                  inb(), inb(), inb(), inb(), inb()],
        out_specs=[outb(), outb(), outb(), outb()],
        out_shape=[jax.ShapeDtypeStruct((bsz * n_iblk, n_tiles, _ROWS),
                                        jnp.float32)] * 4,
        compiler_params=pltpu.CompilerParams(
            dimension_semantics=("parallel", "parallel")),
    )(key_data, resh(logits), resh(xp0), resh(xp1), resh(w), resh(safe_q))

    unb = lambda a: a.reshape(bsz, num_p)
    xr0, xr1, wsel, qsel = unb(xr0), unb(xr1), unb(wsel), unb(qsel)

    log_w_next, vn0, vn1 = pl.pallas_call(
        _stage_c,
        grid=(n_bg,),
        in_specs=[row(), row(), row(), row(), row(), row()],
        out_specs=[row(), row(), row()],
        out_shape=[jax.ShapeDtypeStruct((bsz, num_p), jnp.float32)] * 3,
        compiler_params=pltpu.CompilerParams(
            dimension_semantics=("parallel",)),
    )(wsel, qsel, xr0, xr1, px0, px1)

    x_next = jnp.stack([xr0, xr1], axis=-1)
    v_next = jnp.stack([vn0, vn1], axis=-1)
    x_est = jnp.stack([est0[:, 0], est1[:, 0]], axis=-1)
    return (x_next, v_next, log_w_next, x_est)


# speedup vs baseline: 1.4447x; 1.0150x over previous
import functools
import math

import jax
import jax.numpy as jnp
import numpy as np
from jax.experimental import pallas as pl
from jax.experimental.pallas import tpu as pltpu

_MIN_SCALE = 1e-4
_EPS = 1e-8
_ALPHA = 0.5
_TINY = np.float32(np.finfo(np.float32).tiny)
_LOG2PI = np.float32(np.log(2.0 * np.pi))
_NEG_INF = np.float32(-np.inf)

_I_BLK = 2048
_ROWS = 8
_JW = 512
_UNROLL = 2
_BG = 8


def _tf_round(x0, x1, r):
    x0 = x0 + x1
    x1 = (x1 << np.uint32(r)) | (x1 >> np.uint32(32 - r))
    x1 = x1 ^ x0
    return x0, x1


def _bits_to_gumbel(bits):
    fb = (bits >> np.uint32(9)) | np.uint32(0x3F800000)
    f = jax.lax.bitcast_convert_type(fb, jnp.float32) - np.float32(1.0)
    u = jnp.maximum(_TINY, f + _TINY)
    return -jnp.log(-jnp.log(u))


def _stage_a(px0_ref, px1_ref, v0_ref, v1_ref, n0_ref, n1_ref, lw_ref,
             z_ref, anc_ref, sc_ref,
             oxp0_ref, oxp1_ref, ow_ref, oq_ref, ologit_ref, oest0_ref,
             oest1_ref, *, num_m):
    xp0 = px0_ref[...] + v0_ref[...] + n0_ref[...]
    xp1 = px1_ref[...] + v1_ref[...] + n1_ref[...]
    oxp0_ref[...] = xp0
    oxp1_ref[...] = xp1

    var = sc_ref[0]
    log_var = sc_ref[1]
    acc = jnp.zeros_like(xp0)
    for m in range(num_m):
        dx = xp0 - anc_ref[m, 0]
        dy = xp1 - anc_ref[m, 1]
        ypred = jnp.sqrt(dx * dx + dy * dy)
        zcol = jnp.concatenate(
            [jnp.full((1, 1), z_ref[bb, m], jnp.float32)
             for bb in range(_BG)], axis=0)
        inn = zcol - ypred
        acc = acc + (inn * inn / var + log_var + _LOG2PI)
    log_like = np.float32(-0.5) * acc

    lwu = lw_ref[...] + log_like
    amax = jnp.max(lwu, axis=-1, keepdims=True)
    lse = jnp.log(jnp.sum(jnp.exp(lwu - amax), axis=-1, keepdims=True)) + amax
    log_w = lwu - lse
    w = jnp.exp(log_w)
    ow_ref[...] = w

    num_p = xp0.shape[-1]
    oest0_ref[...] = jnp.broadcast_to(
        jnp.sum(w * xp0, axis=-1, keepdims=True), oest0_ref.shape)
    oest1_ref[...] = jnp.broadcast_to(
        jnp.sum(w * xp1, axis=-1, keepdims=True), oest1_ref.shape)

    q = np.float32(_ALPHA) * w + np.float32((1.0 - _ALPHA) * (1.0 / num_p))
    q_sum = jnp.sum(q, axis=-1, keepdims=True)
    safe_q = q / jnp.maximum(q_sum, np.float32(_EPS))
    safe_q = jnp.maximum(safe_q, np.float32(_EPS))
    safe_q = safe_q / jnp.maximum(
        jnp.sum(safe_q, axis=-1, keepdims=True), np.float32(_EPS))
    oq_ref[...] = safe_q
    ologit_ref[...] = jnp.log(safe_q)


def _stage_b(key_ref, logit_ref, xp0_ref, xp1_ref, w_ref, q_ref,
             oxr0_ref, oxr1_ref, owsel_ref, oqsel_ref, *, num_p, log2_n):
    b = pl.program_id(0)
    ib = pl.program_id(1)
    k0 = key_ref[0]
    k1 = key_ref[1]
    ks2 = k0 ^ k1 ^ np.uint32(0x1BD11BDA)

    n_tiles = _I_BLK // _ROWS
    n_jiters = num_p // _JW
    row0 = (b.astype(jnp.uint32) * np.uint32(num_p)
            + ib.astype(jnp.uint32) * np.uint32(_I_BLK))
    sub_iota = jax.lax.broadcasted_iota(jnp.uint32, (_ROWS, 1), 0)
    lane_iota_u = jax.lax.broadcasted_iota(jnp.uint32, (1, _JW), 1)
    lane_iota_i = jax.lax.broadcasted_iota(jnp.int32, (1, _JW), 1)
    lo_mask = np.uint32((1 << (32 - log2_n)) - 1)
    rot = ((13, 15, 26, 6), (17, 29, 16, 24))
    inj = ((k1, ks2), (ks2, k0), (k0, k1), (k1, ks2), (ks2, k0))

    def tile_body(t, _):
        r = row0 + t.astype(jnp.uint32) * np.uint32(_ROWS) + sub_iota
        hi = r >> np.uint32(32 - log2_n)
        lo_base = (r & lo_mask) << np.uint32(log2_n)
        x0_init = hi + k0
        x1_base = lo_base + k1

        def j_body(cc, carry):
            best, bidx = carry
            scores = []
            for u_i in range(_UNROLL):
                c = cc * _UNROLL + u_i
                jj_u = c.astype(jnp.uint32) * np.uint32(_JW) + lane_iota_u
                x0 = x0_init
                x1 = x1_base + jj_u
                for i in range(5):
                    for rr in rot[i % 2]:
                        x0, x1 = _tf_round(x0, x1, rr)
                    a, bb2 = inj[i]
                    x0 = x0 + a
                    x1 = x1 + bb2 + np.uint32(i + 1)
                bits = x0 ^ x1
                fb = (bits >> np.uint32(9)) | np.uint32(0x3F800000)
                f = jax.lax.bitcast_convert_type(fb, jnp.float32) \
                    - np.float32(1.0)
                uu = jnp.maximum(_TINY, f + _TINY)
                nlu = np.float32(0.0) - jnp.log(uu)
                scores.append(logit_ref[0, c, :].reshape(1, _JW)
                              - jnp.log(nlu))
            for u_i in range(_UNROLL):
                c = cc * _UNROLL + u_i
                score = scores[u_i]
                upd = score > best
                jj_i = c * _JW + lane_iota_i
                best = jnp.maximum(best, score)
                bidx = jnp.where(upd, jnp.broadcast_to(jj_i, upd.shape),
                                 bidx)
            return best, bidx

        init = (jnp.full((_ROWS, _JW), _NEG_INF, jnp.float32),
                jnp.zeros((_ROWS, _JW), jnp.int32))
        best, bidx = jax.lax.fori_loop(0, n_jiters // _UNROLL, j_body, init)

        rowmax = jnp.max(best, axis=1, keepdims=True)
        widx = jnp.min(jnp.where(best == rowmax, bidx, np.int32(2**30)),
                       axis=1, keepdims=True)

        def v_body(c, acc):
            a0, a1, aw, aq = acc
            jj_i = c * _JW + lane_iota_i
            m = jj_i == widx
            a0 = jnp.where(m, xp0_ref[0, c, :].reshape(1, _JW), a0)
            a1 = jnp.where(m, xp1_ref[0, c, :].reshape(1, _JW), a1)
            aw = jnp.where(m, w_ref[0, c, :].reshape(1, _JW), aw)
            aq = jnp.where(m, q_ref[0, c, :].reshape(1, _JW), aq)
            return a0, a1, aw, aq

        z = jnp.zeros((_ROWS, _JW), jnp.float32)
        a0, a1, aw, aq = jax.lax.fori_loop(
            0, n_jiters, v_body, (z, z, z, z))
        oxr0_ref[0, t, :] = jnp.sum(a0, axis=1)
        oxr1_ref[0, t, :] = jnp.sum(a1, axis=1)
        owsel_ref[0, t, :] = jnp.sum(aw, axis=1)
        oqsel_ref[0, t, :] = jnp.sum(aq, axis=1)
        return 0

    jax.lax.fori_loop(0, n_tiles, tile_body, 0)


def _stage_c(wsel_ref, qsel_ref, xr0_ref, xr1_ref, px0_ref, px1_ref,
             olw_ref, ov0_ref, ov1_ref):
    w_corr = wsel_ref[...] / jnp.maximum(qsel_ref[...], np.float32(_EPS))
    lw = jnp.log(jnp.maximum(w_corr, np.float32(_EPS)))
    amax = jnp.max(lw, axis=-1, keepdims=True)
    lse = jnp.log(jnp.sum(jnp.exp(lw - amax), axis=-1, keepdims=True)) + amax
    olw_ref[...] = lw - lse
    ov0_ref[...] = xr0_ref[...] - px0_ref[...]
    ov1_ref[...] = xr1_ref[...] - px1_ref[...]


def kernel(x_prev, v_prev, log_w_prev, z_t, anchors, log_process_scale,
           log_obs_scale):
    bsz, num_p, _ = x_prev.shape
    num_m = z_t.shape[1]
    log2_n = int(math.log2(num_p))
    assert (1 << log2_n) == num_p

    key = jax.random.key(42)
    k_noise, k_res = jax.random.split(key)
    process_scale = jax.nn.softplus(log_process_scale) + _MIN_SCALE
    obs_scale = jax.nn.softplus(log_obs_scale) + _MIN_SCALE
    noise = jax.random.normal(k_noise, x_prev.shape, dtype=x_prev.dtype) \
        * process_scale.reshape(1, 1, -1)
    var = jnp.maximum(obs_scale * obs_scale, _MIN_SCALE)
    sc = jnp.concatenate([var, jnp.log(var)]).astype(jnp.float32)
    key_data = jax.random.key_data(k_res).astype(jnp.uint32)

    px0 = x_prev[:, :, 0]
    px1 = x_prev[:, :, 1]
    n_bg = bsz // _BG
    row = lambda: pl.BlockSpec((_BG, num_p), lambda g: (g, 0))
    est = lambda: pl.BlockSpec((_BG, 128), lambda g: (g, 0))

    xp0, xp1, w, safe_q, logits, est0, est1 = pl.pallas_call(
        functools.partial(_stage_a, num_m=num_m),
        grid=(n_bg,),
        in_specs=[row(), row(), row(), row(), row(), row(), row(),
                  pl.BlockSpec((_BG, num_m), lambda g: (g, 0),
                               memory_space=pltpu.SMEM),
                  pl.BlockSpec((num_m, 2), lambda g: (0, 0),
                               memory_space=pltpu.SMEM),
                  pl.BlockSpec((2,), lambda g: (0,),
                               memory_space=pltpu.SMEM)],
        out_specs=[row(), row(), row(), row(), row(), est(), est()],
        out_shape=[jax.ShapeDtypeStruct((bsz, num_p), jnp.float32)] * 5
        + [jax.ShapeDtypeStruct((bsz, 128), jnp.float32)] * 2,
        compiler_params=pltpu.CompilerParams(
            dimension_semantics=("parallel",)),
    )(px0, px1, v_prev[:, :, 0], v_prev[:, :, 1],
      noise[:, :, 0], noise[:, :, 1], log_w_prev, z_t, anchors, sc)

    n_iblk = num_p // _I_BLK
    n_jiters = num_p // _JW
    n_tiles = _I_BLK // _ROWS
    resh = lambda a: a.reshape(bsz, n_jiters, _JW)
    inb = lambda: pl.BlockSpec((1, n_jiters, _JW), lambda b, ib: (b, 0, 0))
    outb = lambda: pl.BlockSpec((1, n_tiles, _ROWS),
                                lambda b, ib: (b * n_iblk + ib, 0, 0))
    xr0, xr1, wsel, qsel = pl.pallas_call(
        functools.partial(_stage_b, num_p=num_p, log2_n=log2_n),
        grid=(bsz, n_iblk),
        in_specs=[pl.BlockSpec(memory_space=pltpu.SMEM),
                  inb(), inb(), inb(), inb(), inb()],
        out_specs=[outb(), outb(), outb(), outb()],
        out_shape=[jax.ShapeDtypeStruct((bsz * n_iblk, n_tiles, _ROWS),
                                        jnp.float32)] * 4,
        compiler_params=pltpu.CompilerParams(
            dimension_semantics=("parallel", "parallel")),
    )(key_data, resh(logits), resh(xp0), resh(xp1), resh(w), resh(safe_q))

    unb = lambda a: a.reshape(bsz, num_p)
    xr0, xr1, wsel, qsel = unb(xr0), unb(xr1), unb(wsel), unb(qsel)

    log_w_next, vn0, vn1 = pl.pallas_call(
        _stage_c,
        grid=(n_bg,),
        in_specs=[row(), row(), row(), row(), row(), row()],
        out_specs=[row(), row(), row()],
        out_shape=[jax.ShapeDtypeStruct((bsz, num_p), jnp.float32)] * 3,
        compiler_params=pltpu.CompilerParams(
            dimension_semantics=("parallel",)),
    )(wsel, qsel, xr0, xr1, px0, px1)

    x_next = jnp.stack([xr0, xr1], axis=-1)
    v_next = jnp.stack([vn0, vn1], axis=-1)
    x_est = jnp.stack([est0[:, 0], est1[:, 0]], axis=-1)
    return (x_next, v_next, log_w_next, x_est)
